# baseline (speedup 1.0000x reference)
import functools

import jax
import jax.numpy as jnp
from jax import lax
from jax.experimental import pallas as pl
from jax.experimental.pallas import tpu as pltpu
from jax.experimental.pallas import tpu_sc as plsc

_D = 32
_CHUNK = 128
_J = 50
_IT = 128


@functools.lru_cache(maxsize=None)
def _make_gather(num_rows: int, b_total: int):
    info = plsc.get_sparse_core_info()
    nw = info.num_cores * info.num_subcores
    itw = _IT // nw
    nblk = _J * itw
    assert b_total == nw * nblk * _CHUNK

    mesh = plsc.VectorSubcoreMesh(core_axis_name="c", subcore_axis_name="s")

    @functools.partial(
        pl.kernel,
        mesh=mesh,
        out_type=jax.ShapeDtypeStruct((b_total * _D,), jnp.float32),
        compiler_params=pltpu.CompilerParams(
            use_tc_tiling_on_sc=False, needs_layout_passes=False
        ),
        scratch_types=[
            pltpu.VMEM((nblk, _CHUNK), jnp.int32),
            pltpu.VMEM((2 * itw, _CHUNK, _D), jnp.float32),
            pltpu.VMEM((2 * itw * _CHUNK * _D,), jnp.float32),
            pltpu.SemaphoreType.DMA,
            pltpu.SemaphoreType.DMA,
        ],
    )
    def body(idx_hbm, table_hbm, out_hbm, idx_v, rows_v, jbuf, gsem, wsem):
        c = lax.axis_index("c")
        s = lax.axis_index("s")
        wid = s * info.num_cores + c

        pltpu.sync_copy(idx_hbm.at[wid], idx_v)

        lane_d = [lax.iota(jnp.int32, 16) + 16 * h for h in range(2)]
        svecs = [
            [(d // 8) * (itw * 1024) + k * 1024 + (d % 8) * _CHUNK
             for d in lane_d]
            for k in range(itw)
        ]

        def start_gather(n, slot):
            pltpu.async_copy(
                table_hbm.at[idx_v.at[n]], rows_v.at[slot], gsem
            )

        nring = 2 * itw
        for n in range(nring):
            start_gather(n, n)

        def jgroup(j2, carry):
            for p in range(2):
                jj = j2 * 2 + p

                @pl.when(jj >= 2)
                def _():
                    pltpu.make_async_copy(
                        jbuf.at[pl.ds(p * 16384, 16384)],
                        out_hbm.at[pl.ds(0, itw * _CHUNK * _D)],
                        wsem,
                    ).wait()

                for k in range(itw):
                    n = jj * itw + k
                    slot = p * itw + k

                    pltpu.make_async_copy(
                        table_hbm.at[pl.ds(0, _CHUNK)], rows_v.at[slot], gsem
                    ).wait()

                    def trans(t, carry2):
                        for u in range(32):
                            ii = t * 32 + u
                            for h in range(2):
                                v = rows_v[slot, ii, pl.ds(h * 16, 16)]
                                plsc.store_scatter(
                                    jbuf,
                                    [svecs[k][h] + (p * 16384 + ii)], v
                                )
                        return carry2

                    lax.fori_loop(0, _CHUNK // 32, trans, 0)

                    @pl.when(n + nring < nblk)
                    def _():
                        start_gather(n + nring, slot)

                for dt in range(4):
                    off = ((jj * 4 + dt) * _IT + itw * wid) * (8 * _CHUNK)
                    pltpu.async_copy(
                        jbuf.at[pl.ds(p * 16384 + dt * itw * 8 * _CHUNK,
                                      itw * 8 * _CHUNK)],
                        out_hbm.at[pl.ds(off, itw * 8 * _CHUNK)],
                        wsem,
                    )
            return carry

        lax.fori_loop(0, _J // 2, jgroup, 0)

        for p in range(2):
            pltpu.make_async_copy(
                jbuf.at[pl.ds(p * 16384, 16384)],
                out_hbm.at[pl.ds(0, itw * _CHUNK * _D)], wsem
            ).wait()

    return body


def kernel(x, table):
    b_total = x.shape[0] * x.shape[1]
    idx = x.T.astype(jnp.int32)
    info = plsc.get_sparse_core_info()
    nw = info.num_cores * info.num_subcores
    itw = _IT // nw
    idx3 = (idx.reshape(_J, nw, itw, _CHUNK)
            .transpose(1, 0, 2, 3)
            .reshape(nw, _J * itw, _CHUNK))
    flat = _make_gather(table.shape[0], b_total)(idx3, table)
    out5 = flat.reshape(_J, 4, _IT, 8, _CHUNK)
    return out5.transpose(2, 4, 0, 1, 3).reshape(x.shape + (_D,))


# speedup vs baseline: 1.1651x; 1.1651x over previous
import functools

import jax
import jax.numpy as jnp
from jax import lax
from jax.experimental import pallas as pl
from jax.experimental.pallas import tpu as pltpu
from jax.experimental.pallas import tpu_sc as plsc

_D = 32
_CHUNK = 128
_J = 50
_IT = 128


@functools.lru_cache(maxsize=None)
def _make_gather(num_rows: int, b_total: int):
    info = plsc.get_sparse_core_info()
    nw = info.num_cores * info.num_subcores
    itw = _IT // nw
    nblk = _J * itw
    assert b_total == nw * nblk * _CHUNK

    mesh = plsc.VectorSubcoreMesh(core_axis_name="c", subcore_axis_name="s")

    @functools.partial(
        pl.kernel,
        mesh=mesh,
        out_type=jax.ShapeDtypeStruct((b_total * _D,), jnp.float32),
        compiler_params=pltpu.CompilerParams(
            use_tc_tiling_on_sc=False, needs_layout_passes=False
        ),
        scratch_types=[
            pltpu.VMEM((nblk, _CHUNK), jnp.int32),
            pltpu.VMEM((2 * itw, _CHUNK, _D), jnp.float32),
            pltpu.VMEM((2 * itw * _CHUNK * _D,), jnp.float32),
            pltpu.SemaphoreType.DMA,
            pltpu.SemaphoreType.DMA,
        ],
    )
    def body(idx_hbm, table_hbm, out_hbm, idx_v, rows_v, jbuf, gsem, wsem):
        c = lax.axis_index("c")
        s = lax.axis_index("s")
        wid = s * info.num_cores + c

        pltpu.sync_copy(idx_hbm.at[wid], idx_v)

        lane_d = [lax.iota(jnp.int32, 16) + 16 * h for h in range(2)]
        svecs = [
            [(d // 8) * (itw * 1024) + k * 1024 + (d % 8) * _CHUNK
             for d in lane_d]
            for k in range(itw)
        ]

        def start_gather(n, slot):
            pltpu.async_copy(
                table_hbm.at[idx_v.at[n]], rows_v.at[slot], gsem
            )

        nring = 2 * itw
        for n in range(nring):
            start_gather(n, n)

        def jgroup(j2, carry):
            for p in range(2):
                jj = j2 * 2 + p

                @pl.when(jj >= 2)
                def _():
                    pltpu.make_async_copy(
                        jbuf.at[pl.ds(p * 16384, 16384)],
                        out_hbm.at[pl.ds(0, itw * _CHUNK * _D)],
                        wsem,
                    ).wait()

                for k in range(itw):
                    n = jj * itw + k
                    slot = p * itw + k

                    pltpu.make_async_copy(
                        table_hbm.at[pl.ds(0, _CHUNK)], rows_v.at[slot], gsem
                    ).wait()

                    @plsc.parallel_loop(0, _CHUNK, 8, unroll=2)
                    def trans(t):
                        for u in range(8):
                            ii = t + u
                            for h in range(2):
                                v = rows_v[slot, ii, pl.ds(h * 16, 16)]
                                plsc.store_scatter(
                                    jbuf,
                                    [svecs[k][h] + (p * 16384 + ii)], v
                                )

                    @pl.when(n + nring < nblk)
                    def _():
                        start_gather(n + nring, slot)

                for dt in range(4):
                    off = ((jj * 4 + dt) * _IT + itw * wid) * (8 * _CHUNK)
                    pltpu.async_copy(
                        jbuf.at[pl.ds(p * 16384 + dt * itw * 8 * _CHUNK,
                                      itw * 8 * _CHUNK)],
                        out_hbm.at[pl.ds(off, itw * 8 * _CHUNK)],
                        wsem,
                    )
            return carry

        lax.fori_loop(0, _J // 2, jgroup, 0)

        for p in range(2):
            pltpu.make_async_copy(
                jbuf.at[pl.ds(p * 16384, 16384)],
                out_hbm.at[pl.ds(0, itw * _CHUNK * _D)], wsem
            ).wait()

    return body


def kernel(x, table):
    b_total = x.shape[0] * x.shape[1]
    idx = x.T.astype(jnp.int32)
    info = plsc.get_sparse_core_info()
    nw = info.num_cores * info.num_subcores
    itw = _IT // nw
    idx3 = (idx.reshape(_J, nw, itw, _CHUNK)
            .transpose(1, 0, 2, 3)
            .reshape(nw, _J * itw, _CHUNK))
    flat = _make_gather(table.shape[0], b_total)(idx3, table)
    out5 = flat.reshape(_J, 4, _IT, 8, _CHUNK)
    return out5.transpose(2, 4, 0, 1, 3).reshape(x.shape + (_D,))


# speedup vs baseline: 1.6118x; 1.3834x over previous
import functools

import jax
import jax.numpy as jnp
from jax import lax
from jax.experimental import pallas as pl
from jax.experimental.pallas import tpu as pltpu
from jax.experimental.pallas import tpu_sc as plsc

_D = 32
_CHUNK = 128
_J = 50
_IT = 128
_STRIDE = 136


@functools.lru_cache(maxsize=None)
def _make_gather(num_rows: int, b_total: int):
    info = plsc.get_sparse_core_info()
    nw = info.num_cores * info.num_subcores
    itw = _IT // nw
    nblk = _J * itw
    assert b_total == nw * nblk * _CHUNK
    nseg = b_total * _D // (8 * _CHUNK)

    mesh = plsc.VectorSubcoreMesh(core_axis_name="c", subcore_axis_name="s")

    @functools.partial(
        pl.kernel,
        mesh=mesh,
        out_type=jax.ShapeDtypeStruct((nseg, 8, _CHUNK), jnp.float32),
        compiler_params=pltpu.CompilerParams(
            use_tc_tiling_on_sc=False, needs_layout_passes=False
        ),
        scratch_types=[
            pltpu.VMEM((nblk, _CHUNK), jnp.int32),
            pltpu.VMEM((2 * itw, _CHUNK, _D), jnp.float32),
            pltpu.VMEM((2 * itw, _D, _STRIDE), jnp.float32),
            pltpu.SemaphoreType.DMA,
            pltpu.SemaphoreType.DMA,
        ],
    )
    def body(idx_hbm, table_hbm, out_hbm, idx_v, rows_v, jbuf, gsem, wsem):
        c = lax.axis_index("c")
        s = lax.axis_index("s")
        wid = s * info.num_cores + c

        pltpu.sync_copy(idx_hbm.at[wid], idx_v)

        lane_d = [lax.iota(jnp.int32, 16) + 16 * h for h in range(2)]
        zero16 = jnp.zeros((16,), jnp.int32)
        slot_vec = [zero16 + q for q in range(2 * itw)]

        def start_gather(n, slot):
            pltpu.async_copy(
                table_hbm.at[idx_v.at[n]], rows_v.at[slot], gsem
            )

        nring = 2 * itw
        for n in range(nring):
            start_gather(n, n)

        def jgroup(j2, carry):
            for p in range(2):
                jj = j2 * 2 + p

                @pl.when(jj >= 2)
                def _():
                    for _q in range(itw * 4):
                        pltpu.make_async_copy(
                            jbuf.at[0, pl.ds(0, 8), pl.ds(0, _CHUNK)],
                            out_hbm.at[0],
                            wsem,
                        ).wait()

                for k in range(itw):
                    n = jj * itw + k
                    slot = p * itw + k

                    pltpu.make_async_copy(
                        table_hbm.at[pl.ds(0, _CHUNK)], rows_v.at[slot], gsem
                    ).wait()

                    @plsc.parallel_loop(0, _CHUNK, 8, unroll=2)
                    def trans(t):
                        for u in range(8):
                            ii = t + u
                            for h in range(2):
                                v = rows_v[slot, ii, pl.ds(h * 16, 16)]
                                plsc.store_scatter(
                                    jbuf,
                                    [slot_vec[slot], lane_d[h],
                                     zero16 + ii],
                                    v,
                                )

                    @pl.when(n + nring < nblk)
                    def _():
                        start_gather(n + nring, slot)

                for k in range(itw):
                    for dt in range(4):
                        seg = (jj * 4 + dt) * _IT + itw * wid + k
                        pltpu.async_copy(
                            jbuf.at[p * itw + k, pl.ds(dt * 8, 8),
                                    pl.ds(0, _CHUNK)],
                            out_hbm.at[seg],
                            wsem,
                        )
            return carry

        lax.fori_loop(0, _J // 2, jgroup, 0)

        for _q in range(2 * itw * 4):
            pltpu.make_async_copy(
                jbuf.at[0, pl.ds(0, 8), pl.ds(0, _CHUNK)], out_hbm.at[0], wsem
            ).wait()

    return body


def kernel(x, table):
    b_total = x.shape[0] * x.shape[1]
    idx = x.T.astype(jnp.int32)
    info = plsc.get_sparse_core_info()
    nw = info.num_cores * info.num_subcores
    itw = _IT // nw
    idx3 = (idx.reshape(_J, nw, itw, _CHUNK)
            .transpose(1, 0, 2, 3)
            .reshape(nw, _J * itw, _CHUNK))
    seg = _make_gather(table.shape[0], b_total)(idx3, table)
    out5 = seg.reshape(_J, 4, _IT, 8, _CHUNK)
    return out5.transpose(2, 4, 0, 1, 3).reshape(x.shape + (_D,))
